# baseline (speedup 1.0000x reference)
import functools

import jax
import jax.numpy as jnp
from jax import lax
from jax.experimental import pallas as pl
from jax.experimental.pallas import tpu as pltpu
from jax.experimental.pallas import tpu_sc as plsc

N = 10000
E = 320000
D = 128
H = 32
G = 64
PE = 0.1
PF = 0.1

NC = 2
NS = 16
CHW = 128
CH = -(-E // (NS * CHW))
EPT = CH * CHW
NPAD = 12288
PADR = NPAD - N
NPT = NPAD // NS
NGT = N // NS


def _deg_body(dstdeg_hbm, counts_hbm, idxv, onesv, zv, accsh, dsem):
    c = lax.axis_index("c")
    s = lax.axis_index("s")
    for i in range(CHW // 16):
        onesv[pl.ds(i * 16, 16)] = jnp.ones((16,), jnp.float32)

    def zero_fill(i, carry):
        zv[pl.ds(i * 16, 16)] = jnp.zeros((16,), jnp.float32)
        return carry

    lax.fori_loop(0, NPT // 16, zero_fill, 0)
    pltpu.sync_copy(zv, accsh.at[pl.ds(s * NPT, NPT)])
    pltpu.sync_copy(dstdeg_hbm.at[c, s], idxv)
    plsc.subcore_barrier()

    def body(j, carry):
        pltpu.async_copy(onesv, accsh.at[idxv.at[j]], dsem, add=True)

        @pl.when(j >= 8)
        def _():
            pltpu.make_async_copy(onesv, accsh.at[idxv.at[0]], dsem).wait()

        return carry

    lax.fori_loop(0, CH, body, 0)

    def drain(j, carry):
        pltpu.make_async_copy(onesv, accsh.at[idxv.at[0]], dsem).wait()
        return carry

    lax.fori_loop(0, 8, drain, 0)
    plsc.subcore_barrier()
    pltpu.sync_copy(accsh.at[pl.ds(s * NPT, NPT)], zv)
    pltpu.sync_copy(zv, counts_hbm.at[c, pl.ds(s * NPT, NPT)])


def _scat_body(g_hbm, src_hbm, dst_hbm, out_hbm,
               srcv, dstv, rows0, rows1, rows2, rows3, bounce, gsh, accsh,
               g0, g1, g2, g3, s0, s1, s2, s3):
    c = lax.axis_index("c")
    s = lax.axis_index("s")
    rows = (rows0, rows1, rows2, rows3)
    gsem = (g0, g1, g2, g3)
    ssem = (s0, s1, s2, s3)

    def zero_fill(r, carry):
        bounce[r, pl.ds(0, 16)] = jnp.zeros((16,), jnp.float32)
        bounce[r, pl.ds(16, 16)] = jnp.zeros((16,), jnp.float32)
        return carry

    lax.fori_loop(0, NPT, zero_fill, 0)
    pltpu.sync_copy(bounce, accsh.at[pl.ds(s * NPT, NPT)])
    pltpu.sync_copy(g_hbm.at[c, pl.ds(s * NGT, NGT)],
                    gsh.at[pl.ds(s * NGT, NGT)])
    pltpu.sync_copy(src_hbm.at[s], srcv)
    pltpu.sync_copy(dst_hbm.at[c, s], dstv)
    plsc.subcore_barrier()

    def gwait(b):
        pltpu.make_async_copy(gsh.at[srcv.at[0]], rows[b], gsem[b]).wait()

    def swait(b):
        pltpu.make_async_copy(rows[b], accsh.at[dstv.at[0]], ssem[b]).wait()

    for b in range(4):
        pltpu.async_copy(gsh.at[srcv.at[b]], rows[b], gsem[b])

    def body(i, carry):
        j = 4 * i
        for b in range(4):
            gwait(b)
            pltpu.async_copy(rows[b], accsh.at[dstv.at[j + b]], ssem[b],
                             add=True)
        for b in range(4):
            jn = j + 4 + b

            @pl.when(jn < CH)
            def _():
                swait(b)
                pltpu.async_copy(gsh.at[srcv.at[jn]], rows[b], gsem[b])

        return carry

    lax.fori_loop(0, CH // 4, body, 0)
    for b in range(CH % 4):
        j = (CH // 4) * 4 + b
        gwait(b)
        pltpu.async_copy(rows[b], accsh.at[dstv.at[j]], ssem[b], add=True)
    for b in range(4):
        swait(b)
    plsc.subcore_barrier()
    pltpu.sync_copy(accsh.at[pl.ds(s * NPT, NPT)], bounce)
    pltpu.sync_copy(bounce, out_hbm.at[c, pl.ds(s * NPT, NPT)])


def _sc_mesh():
    return plsc.VectorSubcoreMesh(core_axis_name="c", subcore_axis_name="s",
                                  num_cores=NC, num_subcores=NS)


def _deg_call(dstdeg):
    k = pl.kernel(
        _deg_body,
        out_type=jax.ShapeDtypeStruct((NC, NPAD), jnp.float32),
        mesh=_sc_mesh(),
        scratch_types=[
            pltpu.VMEM((CH, CHW), jnp.int32),
            pltpu.VMEM((CHW,), jnp.float32),
            pltpu.VMEM((NPT,), jnp.float32),
            pltpu.VMEM_SHARED((NPAD,), jnp.float32),
            pltpu.SemaphoreType.DMA,
        ],
    )
    return k(dstdeg)


def _scat_call(gs, src, dst):
    k = pl.kernel(
        _scat_body,
        out_type=jax.ShapeDtypeStruct((NC, NPAD, H), jnp.float32),
        mesh=_sc_mesh(),
        compiler_params=pltpu.CompilerParams(use_tc_tiling_on_sc=False),
        scratch_types=[
            pltpu.VMEM((CH, CHW), jnp.int32),
            pltpu.VMEM((CH, CHW), jnp.int32),
            pltpu.VMEM((CHW, H), jnp.float32),
            pltpu.VMEM((CHW, H), jnp.float32),
            pltpu.VMEM((CHW, H), jnp.float32),
            pltpu.VMEM((CHW, H), jnp.float32),
            pltpu.VMEM((NPT, H), jnp.float32),
            pltpu.VMEM_SHARED((N, H), jnp.float32),
            pltpu.VMEM_SHARED((NPAD, H), jnp.float32),
        ] + [pltpu.SemaphoreType.DMA] * 8,
    )
    return k(gs, src, dst)


def _prep_body(x_ref, w0_ref, fm_ref, c1_ref, c2_ref, g_ref, d_ref):
    x = x_ref[...]
    w0 = w0_ref[...]
    d1 = lax.rsqrt(c1_ref[...] + 1.0)
    d2 = lax.rsqrt(c2_ref[...] + 1.0)
    hp1 = jnp.dot(x, w0, preferred_element_type=jnp.float32)
    hp2 = jnp.dot(x, w0 * fm_ref[...], preferred_element_type=jnp.float32)
    g_ref[0, :, :] = hp1 * d1
    g_ref[1, :, :] = hp2 * d2
    d_ref[0, :, :] = d1
    d_ref[1, :, :] = d2


def _bn(a):
    m = jnp.mean(a, axis=0, keepdims=True)
    v = jnp.mean((a - m) ** 2, axis=0, keepdims=True)
    return (a - m) * lax.rsqrt(v + 1e-5)


def _mid_body(s_ref, g_ref, d_ref,
              b_ref, gam_ref, bet_ref, wn_ref, o_ref):
    wn = wn_ref[...]
    d = d_ref[0, :, :]
    a = d * (s_ref[0, 0:N, :] + g_ref[0, :, :]) + b_ref[...]
    h = jnp.maximum(_bn(a) * gam_ref[...] + bet_ref[...], 0.0)
    o_ref[0, :, :] = jnp.dot(h, wn, preferred_element_type=jnp.float32) * d


def _fin_body(s_ref, g_ref, d_ref,
              b_ref, gam_ref, bet_ref, batch_ref,
              pw1_ref, pb1_ref, pw2_ref, pb2_ref, z_ref):
    oh = (lax.broadcasted_iota(jnp.int32, (G, N), 0)
          == batch_ref[...]).astype(jnp.float32)
    d = d_ref[0, :, :]
    a = d * (s_ref[0, 0:N, :] + g_ref[0, :, :]) + b_ref[...]
    h = jnp.maximum(_bn(a) * gam_ref[...] + bet_ref[...], 0.0)
    p = jnp.dot(oh, h, preferred_element_type=jnp.float32)
    q = jnp.maximum(jnp.dot(p, pw1_ref[...], preferred_element_type=jnp.float32)
                    + pb1_ref[...], 0.0)
    z_ref[0, :, :] = (jnp.dot(q, pw2_ref[...], preferred_element_type=jnp.float32)
                      + pb2_ref[...])


def _prep_call(x, w0, fm_col, c1, c2):
    f = pl.pallas_call(
        _prep_body,
        out_shape=[jax.ShapeDtypeStruct((NC, N, H), jnp.float32),
                   jax.ShapeDtypeStruct((NC, N, 1), jnp.float32)],
    )
    return f(x, w0, fm_col, c1, c2)


def _e_spec(shape):
    return pl.BlockSpec((1,) + shape, lambda e: (e,) + (0,) * len(shape))


def _fix_spec(shape):
    return pl.BlockSpec(shape, lambda e: (0,) * len(shape))


def _mid_call(ss, gs, ds, b, gam, bet, wn):
    f = pl.pallas_call(
        _mid_body,
        grid=(NC,),
        in_specs=[_e_spec((NPAD, H)), _e_spec((N, H)), _e_spec((N, 1)),
                  _fix_spec((1, H)), _fix_spec((1, H)), _fix_spec((1, H)),
                  _fix_spec((H, H))],
        out_specs=_e_spec((N, H)),
        out_shape=jax.ShapeDtypeStruct((NC, N, H), jnp.float32),
    )
    return f(ss, gs, ds, b, gam, bet, wn)


def _fin_call(ss, gs, ds, b, gam, bet, batch_row, pw1, pb1, pw2, pb2):
    f = pl.pallas_call(
        _fin_body,
        grid=(NC,),
        in_specs=[_e_spec((NPAD, H)), _e_spec((N, H)), _e_spec((N, 1)),
                  _fix_spec((1, H)), _fix_spec((1, H)), _fix_spec((1, H)),
                  _fix_spec((1, N)),
                  _fix_spec((H, H)), _fix_spec((1, H)),
                  _fix_spec((H, H)), _fix_spec((1, H))],
        out_specs=_e_spec((G, H)),
        out_shape=jax.ShapeDtypeStruct((NC, G, H), jnp.float32),
    )
    z = f(ss, gs, ds, b, gam, bet, batch_row, pw1, pb1, pw2, pb2)
    return z[0], z[1]


def kernel(x, edge_index, batch, W0, b0, W1, b1, W2, b2,
           g0, be0, g1, be1, g2, be2, pW1, pb1, pW2, pb2):
    src = edge_index[0]
    dst = edge_index[1]

    akey = jax.random.key(42)
    k1, k2 = jax.random.split(akey)
    keep = jax.random.bernoulli(k1, 1.0 - PE, (E,))
    fmask = jax.random.bernoulli(k2, 1.0 - PF, (D,)).astype(jnp.float32)

    spread = (jnp.arange(E, dtype=jnp.int32) % PADR) + N
    dst1 = jnp.where(keep, dst, spread)
    padlen = NS * EPT - E
    padsrc = jnp.arange(padlen, dtype=jnp.int32) % N
    paddst = (jnp.arange(padlen, dtype=jnp.int32) % PADR) + N

    def lay(a, pad):
        return jnp.concatenate([a, pad]).reshape(NS, CH, CHW)

    SRC = lay(src, padsrc)
    DST = jnp.stack([lay(dst1, paddst), lay(dst, paddst)])

    counts = _deg_call(DST)
    c1 = counts[0, :N, None]
    c2 = counts[1, :N, None]

    fm_col = fmask[:, None]
    batch_row = batch[None, :].astype(jnp.int32)

    def row(v):
        return v[None, :]

    GS, DS = _prep_call(x, W0, fm_col, c1, c2)
    SS = _scat_call(GS, SRC, DST)
    GS = _mid_call(SS, GS, DS, row(b0), row(g0), row(be0), W1)
    SS = _scat_call(GS, SRC, DST)
    GS = _mid_call(SS, GS, DS, row(b1), row(g1), row(be1), W2)
    SS = _scat_call(GS, SRC, DST)
    z1, z2 = _fin_call(SS, GS, DS, row(b2), row(g2), row(be2),
                       batch_row, pW1, row(pb1), pW2, row(pb2))
    return (z1, z2)


# speedup vs baseline: 1.0515x; 1.0515x over previous
import functools

import jax
import jax.numpy as jnp
from jax import lax
from jax.experimental import pallas as pl
from jax.experimental.pallas import tpu as pltpu
from jax.experimental.pallas import tpu_sc as plsc

N = 10000
E = 320000
D = 128
H = 32
G = 64
PE = 0.1
PF = 0.1

NC = 2
NS = 16
CHW = 128
CH = -(-E // (NS * CHW))
EPT = CH * CHW
NPAD = 12288
PADR = NPAD - N
NPT = NPAD // NS
NGT = N // NS


def _deg_body(dstdeg_hbm, counts_hbm, idxv, onesv, zv, accsh, dsem):
    c = lax.axis_index("c")
    s = lax.axis_index("s")
    for i in range(CHW // 16):
        onesv[pl.ds(i * 16, 16)] = jnp.ones((16,), jnp.float32)

    def zero_fill(i, carry):
        zv[pl.ds(i * 16, 16)] = jnp.zeros((16,), jnp.float32)
        return carry

    lax.fori_loop(0, NPT // 16, zero_fill, 0)
    pltpu.sync_copy(zv, accsh.at[pl.ds(s * NPT, NPT)])
    pltpu.sync_copy(dstdeg_hbm.at[c, s], idxv)
    plsc.subcore_barrier()

    def body(j, carry):
        pltpu.async_copy(onesv, accsh.at[idxv.at[j]], dsem, add=True)

        @pl.when(j >= 8)
        def _():
            pltpu.make_async_copy(onesv, accsh.at[idxv.at[0]], dsem).wait()

        return carry

    lax.fori_loop(0, CH, body, 0)

    def drain(j, carry):
        pltpu.make_async_copy(onesv, accsh.at[idxv.at[0]], dsem).wait()
        return carry

    lax.fori_loop(0, 8, drain, 0)
    plsc.subcore_barrier()
    pltpu.sync_copy(accsh.at[pl.ds(s * NPT, NPT)], zv)
    pltpu.sync_copy(zv, counts_hbm.at[c, pl.ds(s * NPT, NPT)])


def _scat_body(g_hbm, src_hbm, dst_hbm, out_hbm,
               srcv, dstv, rows0, rows1, bounce, gsh, accsh,
               sem0, sem1):
    c = lax.axis_index("c")
    s = lax.axis_index("s")

    def zero_fill(r, carry):
        bounce[r, pl.ds(0, 16)] = jnp.zeros((16,), jnp.float32)
        bounce[r, pl.ds(16, 16)] = jnp.zeros((16,), jnp.float32)
        return carry

    lax.fori_loop(0, NPT, zero_fill, 0)
    pltpu.sync_copy(bounce, accsh.at[pl.ds(s * NPT, NPT)])
    pltpu.sync_copy(g_hbm.at[c, pl.ds(s * NGT, NGT)],
                    gsh.at[pl.ds(s * NGT, NGT)])
    pltpu.sync_copy(src_hbm.at[s], srcv)
    pltpu.sync_copy(dst_hbm.at[c, s], dstv)
    plsc.subcore_barrier()

    def wait0():
        pltpu.make_async_copy(gsh.at[srcv.at[0]], rows0, sem0).wait()

    def wait1():
        pltpu.make_async_copy(gsh.at[srcv.at[0]], rows1, sem1).wait()

    pltpu.async_copy(gsh.at[srcv.at[0]], rows0, sem0)

    def body(i, carry):
        j0 = 2 * i
        wait0()
        pltpu.async_copy(gsh.at[srcv.at[j0 + 1]], rows1, sem1)
        pltpu.sync_copy(rows0, accsh.at[dstv.at[j0]], add=True)
        wait1()

        @pl.when(j0 + 2 < CH)
        def _():
            pltpu.async_copy(gsh.at[srcv.at[j0 + 2]], rows0, sem0)

        pltpu.sync_copy(rows1, accsh.at[dstv.at[j0 + 1]], add=True)
        return carry

    lax.fori_loop(0, CH // 2, body, 0)
    if CH % 2:
        wait0()
        pltpu.sync_copy(rows0, accsh.at[dstv.at[CH - 1]], add=True)
    plsc.subcore_barrier()
    pltpu.sync_copy(accsh.at[pl.ds(s * NPT, NPT)], bounce)
    pltpu.sync_copy(bounce, out_hbm.at[c, pl.ds(s * NPT, NPT)])


def _sc_mesh():
    return plsc.VectorSubcoreMesh(core_axis_name="c", subcore_axis_name="s",
                                  num_cores=NC, num_subcores=NS)


def _deg_call(dstdeg):
    k = pl.kernel(
        _deg_body,
        out_type=jax.ShapeDtypeStruct((NC, NPAD), jnp.float32),
        mesh=_sc_mesh(),
        scratch_types=[
            pltpu.VMEM((CH, CHW), jnp.int32),
            pltpu.VMEM((CHW,), jnp.float32),
            pltpu.VMEM((NPT,), jnp.float32),
            pltpu.VMEM_SHARED((NPAD,), jnp.float32),
            pltpu.SemaphoreType.DMA,
        ],
    )
    return k(dstdeg)


def _scat_call(gs, src, dst):
    k = pl.kernel(
        _scat_body,
        out_type=jax.ShapeDtypeStruct((NC, NPAD, H), jnp.float32),
        mesh=_sc_mesh(),
        compiler_params=pltpu.CompilerParams(use_tc_tiling_on_sc=False),
        scratch_types=[
            pltpu.VMEM((CH, CHW), jnp.int32),
            pltpu.VMEM((CH, CHW), jnp.int32),
            pltpu.VMEM((CHW, H), jnp.float32),
            pltpu.VMEM((CHW, H), jnp.float32),
            pltpu.VMEM((NPT, H), jnp.float32),
            pltpu.VMEM_SHARED((N, H), jnp.float32),
            pltpu.VMEM_SHARED((NPAD, H), jnp.float32),
            pltpu.SemaphoreType.DMA,
            pltpu.SemaphoreType.DMA,
        ],
    )
    return k(gs, src, dst)


def _mm_body(x_ref, w0_ref, fm_ref, hp_ref):
    x = x_ref[...]
    w0 = w0_ref[...]
    hp_ref[0, :, :] = jnp.dot(x, w0, preferred_element_type=jnp.float32)
    hp_ref[1, :, :] = jnp.dot(x, w0 * fm_ref[...],
                              preferred_element_type=jnp.float32)


def _gscale_body(hp_ref, c1_ref, c2_ref, g_ref, d_ref):
    d1 = lax.rsqrt(c1_ref[...] + 1.0)
    d2 = lax.rsqrt(c2_ref[...] + 1.0)
    g_ref[0, :, :] = hp_ref[0, :, :] * d1
    g_ref[1, :, :] = hp_ref[1, :, :] * d2
    d_ref[0, :, :] = d1
    d_ref[1, :, :] = d2


def _bn(a):
    m = jnp.mean(a, axis=0, keepdims=True)
    v = jnp.mean((a - m) ** 2, axis=0, keepdims=True)
    return (a - m) * lax.rsqrt(v + 1e-5)


def _mid_body(s_ref, g_ref, d_ref,
              b_ref, gam_ref, bet_ref, wn_ref, o_ref):
    wn = wn_ref[...]
    d = d_ref[0, :, :]
    a = d * (s_ref[0, 0:N, :] + g_ref[0, :, :]) + b_ref[...]
    h = jnp.maximum(_bn(a) * gam_ref[...] + bet_ref[...], 0.0)
    o_ref[0, :, :] = jnp.dot(h, wn, preferred_element_type=jnp.float32) * d


def _fin_body(s_ref, g_ref, d_ref,
              b_ref, gam_ref, bet_ref, batch_ref,
              pw1_ref, pb1_ref, pw2_ref, pb2_ref, z_ref):
    oh = (lax.broadcasted_iota(jnp.int32, (G, N), 0)
          == batch_ref[...]).astype(jnp.float32)
    d = d_ref[0, :, :]
    a = d * (s_ref[0, 0:N, :] + g_ref[0, :, :]) + b_ref[...]
    h = jnp.maximum(_bn(a) * gam_ref[...] + bet_ref[...], 0.0)
    p = jnp.dot(oh, h, preferred_element_type=jnp.float32)
    q = jnp.maximum(jnp.dot(p, pw1_ref[...], preferred_element_type=jnp.float32)
                    + pb1_ref[...], 0.0)
    z_ref[0, :, :] = (jnp.dot(q, pw2_ref[...], preferred_element_type=jnp.float32)
                      + pb2_ref[...])


def _mm_call(x, w0, fm_col):
    f = pl.pallas_call(
        _mm_body,
        out_shape=jax.ShapeDtypeStruct((NC, N, H), jnp.float32),
    )
    return f(x, w0, fm_col)


def _gscale_call(hp, c1, c2):
    f = pl.pallas_call(
        _gscale_body,
        out_shape=[jax.ShapeDtypeStruct((NC, N, H), jnp.float32),
                   jax.ShapeDtypeStruct((NC, N, 1), jnp.float32)],
    )
    return f(hp, c1, c2)


def _e_spec(shape):
    return pl.BlockSpec((1,) + shape, lambda e: (e,) + (0,) * len(shape))


def _fix_spec(shape):
    return pl.BlockSpec(shape, lambda e: (0,) * len(shape))


def _mid_call(ss, gs, ds, b, gam, bet, wn):
    f = pl.pallas_call(
        _mid_body,
        grid=(NC,),
        in_specs=[_e_spec((NPAD, H)), _e_spec((N, H)), _e_spec((N, 1)),
                  _fix_spec((1, H)), _fix_spec((1, H)), _fix_spec((1, H)),
                  _fix_spec((H, H))],
        out_specs=_e_spec((N, H)),
        out_shape=jax.ShapeDtypeStruct((NC, N, H), jnp.float32),
    )
    return f(ss, gs, ds, b, gam, bet, wn)


def _fin_call(ss, gs, ds, b, gam, bet, batch_row, pw1, pb1, pw2, pb2):
    f = pl.pallas_call(
        _fin_body,
        grid=(NC,),
        in_specs=[_e_spec((NPAD, H)), _e_spec((N, H)), _e_spec((N, 1)),
                  _fix_spec((1, H)), _fix_spec((1, H)), _fix_spec((1, H)),
                  _fix_spec((1, N)),
                  _fix_spec((H, H)), _fix_spec((1, H)),
                  _fix_spec((H, H)), _fix_spec((1, H))],
        out_specs=_e_spec((G, H)),
        out_shape=jax.ShapeDtypeStruct((NC, G, H), jnp.float32),
    )
    z = f(ss, gs, ds, b, gam, bet, batch_row, pw1, pb1, pw2, pb2)
    return z[0], z[1]


def kernel(x, edge_index, batch, W0, b0, W1, b1, W2, b2,
           g0, be0, g1, be1, g2, be2, pW1, pb1, pW2, pb2):
    src = edge_index[0]
    dst = edge_index[1]

    akey = jax.random.key(42)
    k1, k2 = jax.random.split(akey)
    keep = jax.random.bernoulli(k1, 1.0 - PE, (E,))
    fmask = jax.random.bernoulli(k2, 1.0 - PF, (D,)).astype(jnp.float32)

    spread = (jnp.arange(E, dtype=jnp.int32) % PADR) + N
    dst1 = jnp.where(keep, dst, spread)
    padlen = NS * EPT - E
    padsrc = jnp.arange(padlen, dtype=jnp.int32) % N
    paddst = (jnp.arange(padlen, dtype=jnp.int32) % PADR) + N

    def lay(a, pad):
        return jnp.concatenate([a, pad]).reshape(NS, CH, CHW)

    SRC = lay(src, padsrc)
    DST = jnp.stack([lay(dst1, paddst), lay(dst, paddst)])

    fm_col = fmask[:, None]
    batch_row = batch[None, :].astype(jnp.int32)

    def row(v):
        return v[None, :]

    HP = _mm_call(x, W0, fm_col)
    counts = _deg_call(DST)
    c1 = counts[0, :N, None]
    c2 = counts[1, :N, None]
    GS, DS = _gscale_call(HP, c1, c2)
    SS = _scat_call(GS, SRC, DST)
    GS = _mid_call(SS, GS, DS, row(b0), row(g0), row(be0), W1)
    SS = _scat_call(GS, SRC, DST)
    GS = _mid_call(SS, GS, DS, row(b1), row(g1), row(be1), W2)
    SS = _scat_call(GS, SRC, DST)
    z1, z2 = _fin_call(SS, GS, DS, row(b2), row(g2), row(be2),
                       batch_row, pW1, row(pb1), pW2, row(pb2))
    return (z1, z2)


# speedup vs baseline: 1.3145x; 1.2501x over previous
import functools

import jax
import jax.numpy as jnp
from jax import lax
from jax.experimental import pallas as pl
from jax.experimental.pallas import tpu as pltpu
from jax.experimental.pallas import tpu_sc as plsc

N = 10000
E = 320000
D = 128
H = 32
G = 64
PE = 0.1
PF = 0.1

NC = 2
NS = 16
CHW = 128
CH = -(-E // (NS * CHW))
EPT = CH * CHW
NPP = 10016
NP4 = NPP // 4
NPL = N // 4
NPAD = 12288
NP4A = NPAD // 4
PADR = NPAD - NPP
NPT = NPAD // NS
NGT = NPP // NS
LW = 4 * H


def _deg_body(dstdeg_hbm, counts_hbm, idxv, onesv, zv, accsh, dsem):
    c = lax.axis_index("c")
    s = lax.axis_index("s")
    for i in range(CHW // 16):
        onesv[pl.ds(i * 16, 16)] = jnp.ones((16,), jnp.float32)

    def zero_fill(i, carry):
        zv[pl.ds(i * 16, 16)] = jnp.zeros((16,), jnp.float32)
        return carry

    lax.fori_loop(0, NPT // 16, zero_fill, 0)
    pltpu.sync_copy(zv, accsh.at[pl.ds(s * NPT, NPT)])
    pltpu.sync_copy(dstdeg_hbm.at[c, s], idxv)
    plsc.subcore_barrier()

    def body(j, carry):
        pltpu.async_copy(onesv, accsh.at[idxv.at[j]], dsem, add=True)

        @pl.when(j >= 8)
        def _():
            pltpu.make_async_copy(onesv, accsh.at[idxv.at[0]], dsem).wait()

        return carry

    lax.fori_loop(0, CH, body, 0)

    def drain(j, carry):
        pltpu.make_async_copy(onesv, accsh.at[idxv.at[0]], dsem).wait()
        return carry

    lax.fori_loop(0, 8, drain, 0)
    plsc.subcore_barrier()
    pltpu.sync_copy(accsh.at[pl.ds(s * NPT, NPT)], zv)
    pltpu.sync_copy(zv, counts_hbm.at[c, pl.ds(s * NPT, NPT)])


def _scat_body(g_hbm, src_hbm, dst_hbm, out_hbm,
               srcv, dstv, rows0, rows1, bounce, gsh, accsh,
               sem0, sem1):
    c = lax.axis_index("c")
    s = lax.axis_index("s")

    def zero_fill(r, carry):
        bounce[r, pl.ds(0, 16)] = jnp.zeros((16,), jnp.float32)
        bounce[r, pl.ds(16, 16)] = jnp.zeros((16,), jnp.float32)
        return carry

    lax.fori_loop(0, NPT, zero_fill, 0)
    pltpu.sync_copy(bounce, accsh.at[pl.ds(s * NPT, NPT)])
    pltpu.sync_copy(g_hbm.at[c, pl.ds(s * NGT, NGT)],
                    gsh.at[pl.ds(s * NGT, NGT)])
    pltpu.sync_copy(src_hbm.at[s], srcv)
    pltpu.sync_copy(dst_hbm.at[c, s], dstv)
    plsc.subcore_barrier()

    def wait0():
        pltpu.make_async_copy(gsh.at[srcv.at[0]], rows0, sem0).wait()

    def wait1():
        pltpu.make_async_copy(gsh.at[srcv.at[0]], rows1, sem1).wait()

    pltpu.async_copy(gsh.at[srcv.at[0]], rows0, sem0)

    def body(i, carry):
        j0 = 2 * i
        wait0()
        pltpu.async_copy(gsh.at[srcv.at[j0 + 1]], rows1, sem1)
        pltpu.sync_copy(rows0, accsh.at[dstv.at[j0]], add=True)
        wait1()

        @pl.when(j0 + 2 < CH)
        def _():
            pltpu.async_copy(gsh.at[srcv.at[j0 + 2]], rows0, sem0)

        pltpu.sync_copy(rows1, accsh.at[dstv.at[j0 + 1]], add=True)
        return carry

    lax.fori_loop(0, CH // 2, body, 0)
    if CH % 2:
        wait0()
        pltpu.sync_copy(rows0, accsh.at[dstv.at[CH - 1]], add=True)
    plsc.subcore_barrier()
    pltpu.sync_copy(accsh.at[pl.ds(s * NPT, NPT)], bounce)
    pltpu.sync_copy(bounce, out_hbm.at[c, pl.ds(s * NPT, NPT)])


def _sc_mesh():
    return plsc.VectorSubcoreMesh(core_axis_name="c", subcore_axis_name="s",
                                  num_cores=NC, num_subcores=NS)


def _deg_call(dstdeg):
    k = pl.kernel(
        _deg_body,
        out_type=jax.ShapeDtypeStruct((NC, NPAD), jnp.float32),
        mesh=_sc_mesh(),
        scratch_types=[
            pltpu.VMEM((CH, CHW), jnp.int32),
            pltpu.VMEM((CHW,), jnp.float32),
            pltpu.VMEM((NPT,), jnp.float32),
            pltpu.VMEM_SHARED((NPAD,), jnp.float32),
            pltpu.SemaphoreType.DMA,
        ],
    )
    return k(dstdeg)


def _scat_call(gs, src, dst):
    k = pl.kernel(
        _scat_body,
        out_type=jax.ShapeDtypeStruct((NC, NPAD, H), jnp.float32),
        mesh=_sc_mesh(),
        compiler_params=pltpu.CompilerParams(use_tc_tiling_on_sc=False),
        scratch_types=[
            pltpu.VMEM((CH, CHW), jnp.int32),
            pltpu.VMEM((CH, CHW), jnp.int32),
            pltpu.VMEM((CHW, H), jnp.float32),
            pltpu.VMEM((CHW, H), jnp.float32),
            pltpu.VMEM((NPT, H), jnp.float32),
            pltpu.VMEM_SHARED((NPP, H), jnp.float32),
            pltpu.VMEM_SHARED((NPAD, H), jnp.float32),
            pltpu.SemaphoreType.DMA,
            pltpu.SemaphoreType.DMA,
        ],
    )
    return k(gs, src, dst)


def _mm_body(xp_ref, wb_ref, wbf_ref, hp_ref):
    xp = xp_ref[...]
    hp_ref[0, :, :] = jnp.dot(xp, wb_ref[...],
                              preferred_element_type=jnp.float32)
    hp_ref[1, :, :] = jnp.dot(xp, wbf_ref[...],
                              preferred_element_type=jnp.float32)


def _gscale_body(hp_ref, dp_ref, g_ref):
    g_ref[0, :, :] = hp_ref[0, :, :] * dp_ref[0, :, :]
    g_ref[1, :, :] = hp_ref[1, :, :] * dp_ref[1, :, :]


def _grp(v):
    m = (v[:, 0:H] + v[:, H:2 * H] + v[:, 2 * H:3 * H]
         + v[:, 3 * H:4 * H]) * 0.25
    return jnp.concatenate([m, m, m, m], axis=1)


def _mid_body(s_ref, g_ref, d_ref, b_ref, gam_ref, bet_ref, wb_ref, o_ref):
    d = d_ref[0, :, :]
    a = d * (s_ref[0, 0:NP4, :] + g_ref[0, :, :]) + b_ref[...]
    m = _grp(jnp.mean(a[0:NPL, :], axis=0, keepdims=True))
    c = a - m
    v = _grp(jnp.mean(c[0:NPL, :] ** 2, axis=0, keepdims=True))
    h = jnp.maximum(c * lax.rsqrt(v + 1e-5) * gam_ref[...] + bet_ref[...],
                    0.0)
    o_ref[0, :, :] = jnp.dot(h, wb_ref[...],
                             preferred_element_type=jnp.float32) * d


def _fin_body(s_ref, g_ref, d_ref, b_ref, gam_ref, bet_ref, batch_ref,
              pw1_ref, pb1_ref, pw2_ref, pb2_ref, z_ref):
    d = d_ref[0, 0:NPL, :]
    a = d * (s_ref[0, 0:NPL, :] + g_ref[0, 0:NPL, :]) + b_ref[...]
    m = _grp(jnp.mean(a, axis=0, keepdims=True))
    c = a - m
    v = _grp(jnp.mean(c ** 2, axis=0, keepdims=True))
    h = jnp.maximum(c * lax.rsqrt(v + 1e-5) * gam_ref[...] + bet_ref[...],
                    0.0)
    bt = batch_ref[...]
    iota = lax.broadcasted_iota(jnp.int32, (G, NPL), 0)
    p = jnp.zeros((G, H), jnp.float32)
    for j in range(4):
        ohj = (iota == bt[j:j + 1, :]).astype(jnp.float32)
        p = p + jnp.dot(ohj, h[:, j * H:(j + 1) * H],
                        preferred_element_type=jnp.float32)
    q = jnp.maximum(jnp.dot(p, pw1_ref[...],
                            preferred_element_type=jnp.float32)
                    + pb1_ref[...], 0.0)
    z_ref[0, :, :] = (jnp.dot(q, pw2_ref[...],
                              preferred_element_type=jnp.float32)
                      + pb2_ref[...])


def _mm_call(xp, wb, wbf):
    f = pl.pallas_call(
        _mm_body,
        out_shape=jax.ShapeDtypeStruct((NC, NP4, LW), jnp.float32),
    )
    return f(xp, wb, wbf)


def _gscale_call(hp, dp):
    f = pl.pallas_call(
        _gscale_body,
        out_shape=jax.ShapeDtypeStruct((NC, NP4, LW), jnp.float32),
    )
    return f(hp, dp)


def _e_spec(shape):
    return pl.BlockSpec((1,) + shape, lambda e: (e,) + (0,) * len(shape))


def _fix_spec(shape):
    return pl.BlockSpec(shape, lambda e: (0,) * len(shape))


def _mid_call(sp, gp, dp, b, gam, bet, wb):
    f = pl.pallas_call(
        _mid_body,
        grid=(NC,),
        in_specs=[_e_spec((NP4A, LW)), _e_spec((NP4, LW)), _e_spec((NP4, LW)),
                  _fix_spec((1, LW)), _fix_spec((1, LW)), _fix_spec((1, LW)),
                  _fix_spec((LW, LW))],
        out_specs=_e_spec((NP4, LW)),
        out_shape=jax.ShapeDtypeStruct((NC, NP4, LW), jnp.float32),
    )
    return f(sp, gp, dp, b, gam, bet, wb)


def _fin_call(sp, gp, dp, b, gam, bet, batch4, pw1, pb1, pw2, pb2):
    f = pl.pallas_call(
        _fin_body,
        grid=(NC,),
        in_specs=[_e_spec((NP4A, LW)), _e_spec((NP4, LW)), _e_spec((NP4, LW)),
                  _fix_spec((1, LW)), _fix_spec((1, LW)), _fix_spec((1, LW)),
                  _fix_spec((4, NPL)),
                  _fix_spec((H, H)), _fix_spec((1, H)),
                  _fix_spec((H, H)), _fix_spec((1, H))],
        out_specs=_e_spec((G, H)),
        out_shape=jax.ShapeDtypeStruct((NC, G, H), jnp.float32),
    )
    z = f(sp, gp, dp, b, gam, bet, batch4, pw1, pb1, pw2, pb2)
    return z[0], z[1]


def kernel(x, edge_index, batch, W0, b0, W1, b1, W2, b2,
           g0, be0, g1, be1, g2, be2, pW1, pb1, pW2, pb2):
    src = edge_index[0]
    dst = edge_index[1]

    akey = jax.random.key(42)
    k1, k2 = jax.random.split(akey)
    keep = jax.random.bernoulli(k1, 1.0 - PE, (E,))
    fmask = jax.random.bernoulli(k2, 1.0 - PF, (D,)).astype(jnp.float32)

    spread = (jnp.arange(E, dtype=jnp.int32) % PADR) + NPP
    dst1 = jnp.where(keep, dst, spread)
    padlen = NS * EPT - E
    padsrc = jnp.arange(padlen, dtype=jnp.int32) % N
    paddst = (jnp.arange(padlen, dtype=jnp.int32) % PADR) + NPP

    def lay(a, pad):
        return jnp.concatenate([a, pad]).reshape(NS, CH, CHW)

    SRC = lay(src, padsrc)
    DST = jnp.stack([lay(dst1, paddst), lay(dst, paddst)])

    eye4 = jnp.eye(4, dtype=jnp.float32)
    Wb0 = jnp.kron(eye4, W0)
    Wb0f = jnp.kron(eye4, W0 * fmask[:, None])
    Wb1 = jnp.kron(eye4, W1)
    Wb2 = jnp.kron(eye4, W2)
    xp = jnp.concatenate(
        [x, jnp.zeros((NPP - N, D), jnp.float32)]).reshape(NP4, 4 * D)
    batch4 = batch.reshape(NPL, 4).T.astype(jnp.int32)

    def tile4(v):
        return jnp.tile(v, 4)[None, :]

    HP = _mm_call(xp, Wb0, Wb0f)
    counts = _deg_call(DST)
    dinv = lax.rsqrt(counts[:, :NPP] + 1.0)
    DP = jnp.broadcast_to(dinv[:, :, None], (NC, NPP, H)).reshape(
        NC, NP4, LW)
    GP = _gscale_call(HP, DP)

    def scat(gp):
        ss = _scat_call(gp.reshape(NC, NPP, H), SRC, DST)
        return ss.reshape(NC, NP4A, LW)

    SP = scat(GP)
    GP = _mid_call(SP, GP, DP, tile4(b0), tile4(g0), tile4(be0), Wb1)
    SP = scat(GP)
    GP = _mid_call(SP, GP, DP, tile4(b1), tile4(g1), tile4(be1), Wb2)
    SP = scat(GP)
    z1, z2 = _fin_call(SP, GP, DP, tile4(b2), tile4(g2), tile4(be2),
                       batch4, pW1, pb1[None, :], pW2, pb2[None, :])
    return (z1, z2)


# speedup vs baseline: 1.3639x; 1.0376x over previous
import functools

import jax
import jax.numpy as jnp
import numpy as np
from jax import lax
from jax.experimental import pallas as pl
from jax.experimental.pallas import tpu as pltpu
from jax.experimental.pallas import tpu_sc as plsc

N = 10000
E = 320000
D = 128
H = 32
G = 64
PE = 0.1
PF = 0.1

NC = 2
NS = 16
CHW = 128
CH = -(-E // (NS * CHW))
EPT = CH * CHW
NPP = 10016
NP4 = NPP // 4
NPL = N // 4
NPAD = 12288
NP4A = NPAD // 4
PADR = NPAD - NPP
NPT = NPAD // NS
NGT = NPP // NS
LW = 4 * H
PADLEN = NS * EPT - E

_K1, _K2 = jax.random.split(jax.random.key(42))
KEEP_C = np.asarray(jax.random.bernoulli(_K1, 1.0 - PE, (E,)))
FMASK_C = np.asarray(jax.random.bernoulli(_K2, 1.0 - PF, (D,))
                     ).astype(np.float32)
SPREAD_C = (np.arange(E, dtype=np.int32) % PADR) + NPP
PADSRC_C = np.arange(PADLEN, dtype=np.int32) % N
PADDST_C = (np.arange(PADLEN, dtype=np.int32) % PADR) + NPP


def _deg_body(dstdeg_hbm, counts_hbm, idxv, onesv, zv, accsh, dsem):
    c = lax.axis_index("c")
    s = lax.axis_index("s")
    for i in range(CHW // 16):
        onesv[pl.ds(i * 16, 16)] = jnp.ones((16,), jnp.float32)

    def zero_fill(i, carry):
        zv[pl.ds(i * 16, 16)] = jnp.zeros((16,), jnp.float32)
        return carry

    lax.fori_loop(0, NPT // 16, zero_fill, 0)
    pltpu.sync_copy(zv, accsh.at[pl.ds(s * NPT, NPT)])
    pltpu.sync_copy(dstdeg_hbm.at[c, s], idxv)
    plsc.subcore_barrier()

    def body(j, carry):
        pltpu.async_copy(onesv, accsh.at[idxv.at[j]], dsem, add=True)

        @pl.when(j >= 8)
        def _():
            pltpu.make_async_copy(onesv, accsh.at[idxv.at[0]], dsem).wait()

        return carry

    lax.fori_loop(0, CH, body, 0)

    def drain(j, carry):
        pltpu.make_async_copy(onesv, accsh.at[idxv.at[0]], dsem).wait()
        return carry

    lax.fori_loop(0, 8, drain, 0)
    plsc.subcore_barrier()
    pltpu.sync_copy(accsh.at[pl.ds(s * NPT, NPT)], zv)
    pltpu.sync_copy(zv, counts_hbm.at[c, pl.ds(s * NPT, NPT)])


def _scat_body(g_hbm, src_hbm, dst_hbm, out_hbm,
               srcv, dstv, rows0, rows1, bounce, gsh, accsh,
               sem0, sem1):
    c = lax.axis_index("c")
    s = lax.axis_index("s")

    def zero_fill(r, carry):
        bounce[r, pl.ds(0, 16)] = jnp.zeros((16,), jnp.float32)
        bounce[r, pl.ds(16, 16)] = jnp.zeros((16,), jnp.float32)
        return carry

    lax.fori_loop(0, NPT, zero_fill, 0)
    pltpu.sync_copy(bounce, accsh.at[pl.ds(s * NPT, NPT)])
    pltpu.sync_copy(g_hbm.at[c, pl.ds(s * NGT, NGT)],
                    gsh.at[pl.ds(s * NGT, NGT)])
    pltpu.sync_copy(src_hbm.at[s], srcv)
    pltpu.sync_copy(dst_hbm.at[c, s], dstv)
    plsc.subcore_barrier()

    def wait0():
        pltpu.make_async_copy(gsh.at[srcv.at[0]], rows0, sem0).wait()

    def wait1():
        pltpu.make_async_copy(gsh.at[srcv.at[0]], rows1, sem1).wait()

    pltpu.async_copy(gsh.at[srcv.at[0]], rows0, sem0)

    def body(i, carry):
        j0 = 2 * i
        wait0()
        pltpu.async_copy(gsh.at[srcv.at[j0 + 1]], rows1, sem1)
        pltpu.sync_copy(rows0, accsh.at[dstv.at[j0]], add=True)
        wait1()

        @pl.when(j0 + 2 < CH)
        def _():
            pltpu.async_copy(gsh.at[srcv.at[j0 + 2]], rows0, sem0)

        pltpu.sync_copy(rows1, accsh.at[dstv.at[j0 + 1]], add=True)
        return carry

    lax.fori_loop(0, CH // 2, body, 0)
    if CH % 2:
        wait0()
        pltpu.sync_copy(rows0, accsh.at[dstv.at[CH - 1]], add=True)
    plsc.subcore_barrier()
    pltpu.sync_copy(accsh.at[pl.ds(s * NPT, NPT)], bounce)
    pltpu.sync_copy(bounce, out_hbm.at[c, pl.ds(s * NPT, NPT)])


def _sc_mesh():
    return plsc.VectorSubcoreMesh(core_axis_name="c", subcore_axis_name="s",
                                  num_cores=NC, num_subcores=NS)


def _deg_call(dstdeg):
    k = pl.kernel(
        _deg_body,
        out_type=jax.ShapeDtypeStruct((NC, NPAD), jnp.float32),
        mesh=_sc_mesh(),
        scratch_types=[
            pltpu.VMEM((CH, CHW), jnp.int32),
            pltpu.VMEM((CHW,), jnp.float32),
            pltpu.VMEM((NPT,), jnp.float32),
            pltpu.VMEM_SHARED((NPAD,), jnp.float32),
            pltpu.SemaphoreType.DMA,
        ],
    )
    return k(dstdeg)


def _scat_call(gs, src, dst):
    k = pl.kernel(
        _scat_body,
        out_type=jax.ShapeDtypeStruct((NC, NPAD, H), jnp.float32),
        mesh=_sc_mesh(),
        compiler_params=pltpu.CompilerParams(use_tc_tiling_on_sc=False),
        scratch_types=[
            pltpu.VMEM((CH, CHW), jnp.int32),
            pltpu.VMEM((CH, CHW), jnp.int32),
            pltpu.VMEM((CHW, H), jnp.float32),
            pltpu.VMEM((CHW, H), jnp.float32),
            pltpu.VMEM((NPT, H), jnp.float32),
            pltpu.VMEM_SHARED((NPP, H), jnp.float32),
            pltpu.VMEM_SHARED((NPAD, H), jnp.float32),
            pltpu.SemaphoreType.DMA,
            pltpu.SemaphoreType.DMA,
        ],
    )
    return k(gs, src, dst)


def _mm_body(xp_ref, wb_ref, wbf_ref, hp_ref):
    xp = xp_ref[...]
    hp_ref[0, :, :] = jnp.dot(xp, wb_ref[...],
                              preferred_element_type=jnp.float32)
    hp_ref[1, :, :] = jnp.dot(xp, wbf_ref[...],
                              preferred_element_type=jnp.float32)


def _gscale_body(hp_ref, dp_ref, g_ref):
    g_ref[0, :, :] = hp_ref[0, :, :] * dp_ref[0, :, :]
    g_ref[1, :, :] = hp_ref[1, :, :] * dp_ref[1, :, :]


def _grp(v):
    m = (v[:, 0:H] + v[:, H:2 * H] + v[:, 2 * H:3 * H]
         + v[:, 3 * H:4 * H]) * 0.25
    return jnp.concatenate([m, m, m, m], axis=1)


def _mid_body(s_ref, g_ref, d_ref, b_ref, gam_ref, bet_ref, wb_ref, o_ref):
    d = d_ref[0, :, :]
    a = d * (s_ref[0, 0:NP4, :] + g_ref[0, :, :]) + b_ref[...]
    m = _grp(jnp.mean(a[0:NPL, :], axis=0, keepdims=True))
    c = a - m
    v = _grp(jnp.mean(c[0:NPL, :] ** 2, axis=0, keepdims=True))
    h = jnp.maximum(c * lax.rsqrt(v + 1e-5) * gam_ref[...] + bet_ref[...],
                    0.0)
    o_ref[0, :, :] = jnp.dot(h, wb_ref[...],
                             preferred_element_type=jnp.float32) * d


def _fin_body(s_ref, g_ref, d_ref, b_ref, gam_ref, bet_ref, batch_ref,
              pw1_ref, pb1_ref, pw2_ref, pb2_ref, z_ref):
    d = d_ref[0, 0:NPL, :]
    a = d * (s_ref[0, 0:NPL, :] + g_ref[0, 0:NPL, :]) + b_ref[...]
    m = _grp(jnp.mean(a, axis=0, keepdims=True))
    c = a - m
    v = _grp(jnp.mean(c ** 2, axis=0, keepdims=True))
    h = jnp.maximum(c * lax.rsqrt(v + 1e-5) * gam_ref[...] + bet_ref[...],
                    0.0)
    bt = batch_ref[...]
    iota = lax.broadcasted_iota(jnp.int32, (G, NPL), 0)
    p = jnp.zeros((G, H), jnp.float32)
    for j in range(4):
        ohj = (iota == bt[j:j + 1, :]).astype(jnp.float32)
        p = p + jnp.dot(ohj, h[:, j * H:(j + 1) * H],
                        preferred_element_type=jnp.float32)
    q = jnp.maximum(jnp.dot(p, pw1_ref[...],
                            preferred_element_type=jnp.float32)
                    + pb1_ref[...], 0.0)
    z_ref[0, :, :] = (jnp.dot(q, pw2_ref[...],
                              preferred_element_type=jnp.float32)
                      + pb2_ref[...])


def _mm_call(xp, wb, wbf):
    f = pl.pallas_call(
        _mm_body,
        out_shape=jax.ShapeDtypeStruct((NC, NP4, LW), jnp.float32),
    )
    return f(xp, wb, wbf)


def _gscale_call(hp, dp):
    f = pl.pallas_call(
        _gscale_body,
        out_shape=jax.ShapeDtypeStruct((NC, NP4, LW), jnp.float32),
    )
    return f(hp, dp)


def _e_spec(shape):
    return pl.BlockSpec((1,) + shape, lambda e: (e,) + (0,) * len(shape))


def _fix_spec(shape):
    return pl.BlockSpec(shape, lambda e: (0,) * len(shape))


def _mid_call(sp, gp, dp, b, gam, bet, wb):
    f = pl.pallas_call(
        _mid_body,
        grid=(NC,),
        in_specs=[_e_spec((NP4A, LW)), _e_spec((NP4, LW)), _e_spec((NP4, LW)),
                  _fix_spec((1, LW)), _fix_spec((1, LW)), _fix_spec((1, LW)),
                  _fix_spec((LW, LW))],
        out_specs=_e_spec((NP4, LW)),
        out_shape=jax.ShapeDtypeStruct((NC, NP4, LW), jnp.float32),
    )
    return f(sp, gp, dp, b, gam, bet, wb)


def _fin_call(sp, gp, dp, b, gam, bet, batch4, pw1, pb1, pw2, pb2):
    f = pl.pallas_call(
        _fin_body,
        grid=(NC,),
        in_specs=[_e_spec((NP4A, LW)), _e_spec((NP4, LW)), _e_spec((NP4, LW)),
                  _fix_spec((1, LW)), _fix_spec((1, LW)), _fix_spec((1, LW)),
                  _fix_spec((4, NPL)),
                  _fix_spec((H, H)), _fix_spec((1, H)),
                  _fix_spec((H, H)), _fix_spec((1, H))],
        out_specs=_e_spec((G, H)),
        out_shape=jax.ShapeDtypeStruct((NC, G, H), jnp.float32),
    )
    z = f(sp, gp, dp, b, gam, bet, batch4, pw1, pb1, pw2, pb2)
    return z[0], z[1]


def kernel(x, edge_index, batch, W0, b0, W1, b1, W2, b2,
           g0, be0, g1, be1, g2, be2, pW1, pb1, pW2, pb2):
    src = edge_index[0]
    dst = edge_index[1]

    fmask = jnp.asarray(FMASK_C)
    dst1 = jnp.where(jnp.asarray(KEEP_C), dst, jnp.asarray(SPREAD_C))

    def lay(a, pad):
        return jnp.concatenate([a, jnp.asarray(pad)]).reshape(NS, CH, CHW)

    SRC = lay(src, PADSRC_C)
    DST = jnp.stack([lay(dst1, PADDST_C), lay(dst, PADDST_C)])

    eye4 = jnp.eye(4, dtype=jnp.float32)
    Wb0 = jnp.kron(eye4, W0)
    Wb0f = jnp.kron(eye4, W0 * fmask[:, None])
    Wb1 = jnp.kron(eye4, W1)
    Wb2 = jnp.kron(eye4, W2)
    xp = jnp.concatenate(
        [x, jnp.zeros((NPP - N, D), jnp.float32)]).reshape(NP4, 4 * D)
    batch4 = batch.reshape(NPL, 4).T.astype(jnp.int32)

    def tile4(v):
        return jnp.tile(v, 4)[None, :]

    HP = _mm_call(xp, Wb0, Wb0f)
    counts = _deg_call(DST)
    dinv = lax.rsqrt(counts[:, :NPP] + 1.0)
    DP = jnp.broadcast_to(dinv[:, :, None], (NC, NPP, H)).reshape(
        NC, NP4, LW)
    GP = _gscale_call(HP, DP)

    def scat(gp):
        ss = _scat_call(gp.reshape(NC, NPP, H), SRC, DST)
        return ss.reshape(NC, NP4A, LW)

    SP = scat(GP)
    GP = _mid_call(SP, GP, DP, tile4(b0), tile4(g0), tile4(be0), Wb1)
    SP = scat(GP)
    GP = _mid_call(SP, GP, DP, tile4(b1), tile4(g1), tile4(be1), Wb2)
    SP = scat(GP)
    z1, z2 = _fin_call(SP, GP, DP, tile4(b2), tile4(g2), tile4(be2),
                       batch4, pW1, pb1[None, :], pW2, pb2[None, :])
    return (z1, z2)
